# Initial kernel scaffold; baseline (speedup 1.0000x reference)
import jax
import jax.numpy as jnp
from jax.experimental import pallas as pl


def kernel(x, edge_index, W, b, gamma, beta):
    raise NotImplementedError("write your pallas kernel here")


# speedup vs baseline: 9.1423x; 9.1423x over previous
import functools

import jax
import jax.numpy as jnp
from jax import lax
from jax.experimental import pallas as pl
from jax.experimental.pallas import tpu as pltpu
from jax.experimental.pallas import tpu_sc as plsc

N_NODES = 10000
D = 128
BN_EPS = 1e-5

NC = 2
NS = 16
CHUNK = 128
HIST = 10240
SLICE = HIST // NS
DUMP_ROW = N_NODES

E_PER_TILE = 10240
E_PAD = NC * NS * E_PER_TILE
N_CHUNKS = E_PER_TILE // CHUNK

DEG_TILES = NS
DEG_PER_TILE = E_PAD // DEG_TILES
DEG_CHUNKS = DEG_PER_TILE // CHUNK


def _zero_vec(ref, n):
    def body(i, _):
        ref[pl.ds(i * 16, 16)] = jnp.zeros((16,), jnp.float32)
        return 0
    lax.fori_loop(0, n // 16, body, 0)


def _zero_rows(ref, rows):
    def body(r, _):
        for j in range(8):
            ref[r, pl.ds(j * 16, 16)] = jnp.zeros((16,), jnp.float32)
        return 0
    lax.fori_loop(0, rows, body, 0)


def _deg_body(dst_hbm, out_hbm, idx_v, ones_v, zero_v, hist_sh):
    s = lax.axis_index("s")

    def ones_body(i, _):
        ones_v[pl.ds(i * 16, 16)] = jnp.ones((16,), jnp.float32)
        return 0
    lax.fori_loop(0, CHUNK // 16, ones_body, 0)
    _zero_vec(zero_v, SLICE)
    pltpu.sync_copy(zero_v, hist_sh.at[pl.ds(s * SLICE, SLICE)])
    plsc.subcore_barrier()

    def chunk_body(ci, _):
        base = s * DEG_PER_TILE + ci * CHUNK
        pltpu.sync_copy(dst_hbm.at[pl.ds(base, CHUNK)], idx_v)
        pltpu.sync_copy(ones_v, hist_sh.at[idx_v], add=True)
        return 0
    lax.fori_loop(0, DEG_CHUNKS, chunk_body, 0)

    plsc.subcore_barrier()
    pltpu.sync_copy(hist_sh.at[pl.ds(s * SLICE, SLICE)],
                    out_hbm.at[pl.ds(s * SLICE, SLICE)])


_deg_kernel = pl.kernel(
    _deg_body,
    out_type=jax.ShapeDtypeStruct((HIST,), jnp.float32),
    mesh=plsc.VectorSubcoreMesh(core_axis_name="c", subcore_axis_name="s",
                                num_cores=1),
    scratch_types=[
        pltpu.VMEM((CHUNK,), jnp.int32),
        pltpu.VMEM((CHUNK,), jnp.float32),
        pltpu.VMEM((SLICE,), jnp.float32),
        pltpu.VMEM_SHARED((HIST,), jnp.float32),
    ],
)


def _edge_body(src_hbm, dst_hbm, y_hbm, out_hbm, sidx, didx, rows, agg_sh, sem):
    c = lax.axis_index("c")
    s = lax.axis_index("s")
    wid = c * NS + s

    _zero_rows(rows, CHUNK)
    for k in range(SLICE // CHUNK):
        pltpu.sync_copy(rows, agg_sh.at[pl.ds(s * SLICE + k * CHUNK, CHUNK)])
    plsc.subcore_barrier()

    ebase = wid * E_PER_TILE

    def chunk_body(ci, _):
        base = ebase + ci * CHUNK
        pltpu.sync_copy(src_hbm.at[pl.ds(base, CHUNK)], sidx)
        pltpu.sync_copy(dst_hbm.at[pl.ds(base, CHUNK)], didx)
        pltpu.async_copy(y_hbm.at[sidx], rows, sem).wait()
        pltpu.sync_copy(rows, agg_sh.at[didx], add=True)
        return 0
    lax.fori_loop(0, N_CHUNKS, chunk_body, 0)

    plsc.subcore_barrier()
    pltpu.sync_copy(agg_sh.at[pl.ds(s * SLICE, SLICE)],
                    out_hbm.at[c, pl.ds(s * SLICE, SLICE), :])


_edge_kernel = pl.kernel(
    _edge_body,
    out_type=jax.ShapeDtypeStruct((NC, HIST, D), jnp.float32),
    mesh=plsc.VectorSubcoreMesh(core_axis_name="c", subcore_axis_name="s"),
    scratch_types=[
        pltpu.VMEM((CHUNK,), jnp.int32),
        pltpu.VMEM((CHUNK,), jnp.int32),
        pltpu.VMEM((CHUNK, D), jnp.float32),
        pltpu.VMEM_SHARED((HIST, D), jnp.float32),
        pltpu.SemaphoreType.DMA,
    ],
)


def _prep_body(x_ref, w_ref, deg_ref, y_ref, dis_ref):
    dis = jax.lax.rsqrt(deg_ref[...] + 1.0)
    dis_ref[...] = dis
    xw = jnp.dot(x_ref[...], w_ref[...], preferred_element_type=jnp.float32)
    y_ref[...] = xw * dis


_prep_kernel = pl.pallas_call(
    _prep_body,
    out_shape=(
        jax.ShapeDtypeStruct((N_NODES, D), jnp.float32),
        jax.ShapeDtypeStruct((N_NODES, 1), jnp.float32),
    ),
)


def _final_body(p_ref, y_ref, dis_ref, x_ref, b_ref, g_ref, be_ref, o_ref):
    agg = p_ref[0, :N_NODES, :] + p_ref[1, :N_NODES, :] + y_ref[...]
    h = agg * dis_ref[...] + b_ref[...]
    mean = jnp.mean(h, axis=0, keepdims=True)
    cent = h - mean
    var = jnp.mean(cent * cent, axis=0, keepdims=True)
    bn = cent * jax.lax.rsqrt(var + BN_EPS) * g_ref[...] + be_ref[...]
    o_ref[...] = jnp.maximum(bn, 0.0) + x_ref[...]


_final_kernel = pl.pallas_call(
    _final_body,
    out_shape=jax.ShapeDtypeStruct((N_NODES, D), jnp.float32),
)


@jax.jit
def kernel(x, edge_index, W, b, gamma, beta):
    ei = edge_index.astype(jnp.int32)
    pad = E_PAD - ei.shape[1]
    src = jnp.concatenate([ei[0], jnp.zeros((pad,), jnp.int32)])
    dst = jnp.concatenate([ei[1], jnp.full((pad,), DUMP_ROW, jnp.int32)])

    hist = _deg_kernel(dst)
    deg_col = hist[:N_NODES].reshape(N_NODES, 1)
    y, dis = _prep_kernel(x, W, deg_col)
    parts = _edge_kernel(src, dst, y)
    return _final_kernel(parts, y, dis, x,
                         b.reshape(1, D), gamma.reshape(1, D),
                         beta.reshape(1, D))


# speedup vs baseline: 12.4660x; 1.3635x over previous
import functools

import jax
import jax.numpy as jnp
from jax import lax
from jax.experimental import pallas as pl
from jax.experimental.pallas import tpu as pltpu
from jax.experimental.pallas import tpu_sc as plsc

N_NODES = 10000
D = 128
BN_EPS = 1e-5

NC = 2
NS = 16
CHUNK = 128
HIST = 10240
SLICE = HIST // NS
DUMP_ROW = N_NODES

E_PER_TILE = 10240
E_PAD = NC * NS * E_PER_TILE
N_CHUNKS = E_PER_TILE // CHUNK

DEG_TILES = NS
DEG_PER_TILE = E_PAD // DEG_TILES
DEG_CHUNKS = DEG_PER_TILE // CHUNK


def _zero_vec(ref, n):
    def body(i, _):
        ref[pl.ds(i * 16, 16)] = jnp.zeros((16,), jnp.float32)
        return 0
    lax.fori_loop(0, n // 16, body, 0)


def _zero_rows(ref, rows):
    def body(r, _):
        for j in range(8):
            ref[r, pl.ds(j * 16, 16)] = jnp.zeros((16,), jnp.float32)
        return 0
    lax.fori_loop(0, rows, body, 0)


def _deg_body(dst_hbm, out_hbm, idx_v, ones_v, zero_v, hist_sh, ssem):
    s = lax.axis_index("s")

    def ones_body(i, _):
        ones_v[pl.ds(i * 16, 16)] = jnp.ones((16,), jnp.float32)
        return 0
    lax.fori_loop(0, CHUNK // 16, ones_body, 0)
    _zero_vec(zero_v, SLICE)
    pltpu.sync_copy(zero_v, hist_sh.at[pl.ds(s * SLICE, SLICE)])
    pltpu.sync_copy(dst_hbm.at[s], idx_v)
    plsc.subcore_barrier()

    def chunk_body(ci, _):
        pltpu.async_copy(ones_v, hist_sh.at[idx_v.at[ci]], ssem, add=True)
        return 0
    lax.fori_loop(0, DEG_CHUNKS, chunk_body, 0)

    def drain_body(ci, _):
        pltpu.make_async_copy(ones_v, hist_sh.at[idx_v.at[ci]], ssem).wait()
        return 0
    lax.fori_loop(0, DEG_CHUNKS, drain_body, 0)

    plsc.subcore_barrier()
    pltpu.sync_copy(hist_sh.at[pl.ds(s * SLICE, SLICE)],
                    out_hbm.at[pl.ds(s * SLICE, SLICE)])


_deg_kernel = pl.kernel(
    _deg_body,
    out_type=jax.ShapeDtypeStruct((HIST,), jnp.float32),
    mesh=plsc.VectorSubcoreMesh(core_axis_name="c", subcore_axis_name="s",
                                num_cores=1),
    scratch_types=[
        pltpu.VMEM((DEG_CHUNKS, CHUNK), jnp.int32),
        pltpu.VMEM((CHUNK,), jnp.float32),
        pltpu.VMEM((SLICE,), jnp.float32),
        pltpu.VMEM_SHARED((HIST,), jnp.float32),
        pltpu.SemaphoreType.DMA,
    ],
)


def _edge_body(idx_hbm, y_hbm, out_hbm, idxb, rows, agg_sh,
               isem0, isem1, isem2, isem3, gsem0, gsem1, ssem0, ssem1):
    c = lax.axis_index("c")
    s = lax.axis_index("s")
    wid = c * NS + s
    isems = (isem0, isem1, isem2, isem3)
    gsems = (gsem0, gsem1)
    ssems = (ssem0, ssem1)

    def zero_rows0(r, _):
        for j in range(8):
            rows[0, r, pl.ds(j * 16, 16)] = jnp.zeros((16,), jnp.float32)
        return 0
    lax.fori_loop(0, CHUNK, zero_rows0, 0)
    for k in range(SLICE // CHUNK):
        pltpu.sync_copy(rows.at[0],
                        agg_sh.at[pl.ds(s * SLICE + k * CHUNK, CHUNK)])
    plsc.subcore_barrier()

    def idx_load(ci, ib):
        pltpu.async_copy(idx_hbm.at[wid, ci], idxb.at[ib], isems[ib])

    def wait_idx(ci, ib):
        pltpu.make_async_copy(idx_hbm.at[wid, ci], idxb.at[ib],
                              isems[ib]).wait()

    def gather(ci, ib, b):
        pltpu.async_copy(y_hbm.at[idxb.at[ib, 0]], rows.at[b], gsems[b])

    def wait_gather(ib, b):
        pltpu.make_async_copy(y_hbm.at[idxb.at[ib, 0]], rows.at[b],
                              gsems[b]).wait()

    def scatter(ib, b):
        pltpu.async_copy(rows.at[b], agg_sh.at[idxb.at[ib, 1]], ssems[b],
                         add=True)

    def wait_scatter(ib, b):
        pltpu.make_async_copy(rows.at[b], agg_sh.at[idxb.at[ib, 1]],
                              ssems[b]).wait()

    idx_load(0, 0)
    idx_load(1, 1)

    @pl.loop(0, N_CHUNKS, step=4)
    def quad(g):
        for u in range(4):
            ci = g + u
            b = u % 2
            wait_idx(ci, u)
            if u >= 2:
                wait_scatter(u - 2, b)
            else:
                @pl.when(g > 0)
                def _():
                    wait_scatter(u + 2, b)
            nci = ci + 2
            nib = (u + 2) % 4
            if u < 2:
                idx_load(nci, nib)
            else:
                @pl.when(nci < N_CHUNKS)
                def _():
                    idx_load(nci, nib)
            gather(ci, u, b)
            wait_gather(u, b)
            scatter(u, b)

    wait_scatter(2, 0)
    wait_scatter(3, 1)
    plsc.subcore_barrier()
    pltpu.sync_copy(agg_sh.at[pl.ds(s * SLICE, SLICE)],
                    out_hbm.at[c, pl.ds(s * SLICE, SLICE), :])


_edge_kernel = pl.kernel(
    _edge_body,
    out_type=jax.ShapeDtypeStruct((NC, HIST, D), jnp.float32),
    mesh=plsc.VectorSubcoreMesh(core_axis_name="c", subcore_axis_name="s"),
    scratch_types=[
        pltpu.VMEM((4, 2, CHUNK), jnp.int32),
        pltpu.VMEM((2, CHUNK, D), jnp.float32),
        pltpu.VMEM_SHARED((HIST, D), jnp.float32),
        pltpu.SemaphoreType.DMA,
        pltpu.SemaphoreType.DMA,
        pltpu.SemaphoreType.DMA,
        pltpu.SemaphoreType.DMA,
        pltpu.SemaphoreType.DMA,
        pltpu.SemaphoreType.DMA,
        pltpu.SemaphoreType.DMA,
        pltpu.SemaphoreType.DMA,
    ],
)


def _prep_body(x_ref, w_ref, deg_ref, y_ref, dis_ref):
    dis = jax.lax.rsqrt(deg_ref[...] + 1.0)
    dis_ref[...] = dis
    xw = jnp.dot(x_ref[...], w_ref[...], preferred_element_type=jnp.float32)
    y_ref[...] = xw * dis


_prep_kernel = pl.pallas_call(
    _prep_body,
    out_shape=(
        jax.ShapeDtypeStruct((N_NODES, D), jnp.float32),
        jax.ShapeDtypeStruct((N_NODES, 1), jnp.float32),
    ),
)


def _final_body(p_ref, y_ref, dis_ref, x_ref, b_ref, g_ref, be_ref, o_ref):
    agg = p_ref[0, :N_NODES, :] + p_ref[1, :N_NODES, :] + y_ref[...]
    h = agg * dis_ref[...] + b_ref[...]
    mean = jnp.mean(h, axis=0, keepdims=True)
    cent = h - mean
    var = jnp.mean(cent * cent, axis=0, keepdims=True)
    bn = cent * jax.lax.rsqrt(var + BN_EPS) * g_ref[...] + be_ref[...]
    o_ref[...] = jnp.maximum(bn, 0.0) + x_ref[...]


_final_kernel = pl.pallas_call(
    _final_body,
    out_shape=jax.ShapeDtypeStruct((N_NODES, D), jnp.float32),
)


@jax.jit
def kernel(x, edge_index, W, b, gamma, beta):
    ei = edge_index.astype(jnp.int32)
    pad = E_PAD - ei.shape[1]
    src = jnp.concatenate([ei[0], jnp.zeros((pad,), jnp.int32)])
    dst = jnp.concatenate([ei[1], jnp.full((pad,), DUMP_ROW, jnp.int32)])

    hist = _deg_kernel(dst.reshape(DEG_TILES, DEG_CHUNKS, CHUNK))
    deg_col = hist[:N_NODES].reshape(N_NODES, 1)
    y, dis = _prep_kernel(x, W, deg_col)
    idx4 = jnp.stack([src.reshape(NC * NS, N_CHUNKS, CHUNK),
                      dst.reshape(NC * NS, N_CHUNKS, CHUNK)], axis=2)
    parts = _edge_kernel(idx4, y)
    return _final_kernel(parts, y, dis, x,
                         b.reshape(1, D), gamma.reshape(1, D),
                         beta.reshape(1, D))


# speedup vs baseline: 40.5534x; 3.2531x over previous
import functools

import jax
import jax.numpy as jnp
from jax import lax
from jax.experimental import pallas as pl
from jax.experimental.pallas import tpu as pltpu
from jax.experimental.pallas import tpu_sc as plsc

N_NODES = 10000
D = 128
BN_EPS = 1e-5

NC = 2
NS = 16
CHUNK = 80
HIST = 10240
SLICE = HIST // NS
DUMP_ROW = N_NODES

E_PER_TILE = 10240
E_PAD = NC * NS * E_PER_TILE
N_CHUNKS = E_PER_TILE // CHUNK

N_EDGES = 320000
E_PAD_TAIL = E_PAD - N_EDGES
REAL_LAST = (N_EDGES - (NC * NS - 1) * E_PER_TILE) // CHUNK

DCH = 128
DEG_CPT = E_PAD // (NS * DCH)


def _zero_vec(ref, n):
    def body(i, _):
        ref[pl.ds(i * 16, 16)] = jnp.zeros((16,), jnp.float32)
        return 0
    lax.fori_loop(0, n // 16, body, 0)


def _zero_rows(ref, rows):
    def body(r, _):
        for j in range(8):
            ref[r, pl.ds(j * 16, 16)] = jnp.zeros((16,), jnp.float32)
        return 0
    lax.fori_loop(0, rows, body, 0)


def _deg_body(dst_hbm, out_hbm, idx_v, ones_v, zero_v, hist_sh, ssem):
    s = lax.axis_index("s")

    def ones_body(i, _):
        ones_v[pl.ds(i * 16, 16)] = jnp.ones((16,), jnp.float32)
        return 0
    lax.fori_loop(0, DCH // 16, ones_body, 0)
    _zero_vec(zero_v, SLICE)
    pltpu.sync_copy(zero_v, hist_sh.at[pl.ds(s * SLICE, SLICE)])
    pltpu.sync_copy(dst_hbm.at[s], idx_v)
    plsc.subcore_barrier()

    def chunk_body(ci, _):
        pltpu.async_copy(ones_v, hist_sh.at[idx_v.at[ci]], ssem, add=True)
        return 0
    lax.fori_loop(0, DEG_CPT, chunk_body, 0)

    def drain_body(ci, _):
        pltpu.make_async_copy(ones_v, hist_sh.at[idx_v.at[ci]], ssem).wait()
        return 0
    lax.fori_loop(0, DEG_CPT, drain_body, 0)

    plsc.subcore_barrier()
    pltpu.sync_copy(hist_sh.at[pl.ds(s * SLICE, SLICE)],
                    out_hbm.at[pl.ds(s * SLICE, SLICE)])


_deg_kernel = pl.kernel(
    _deg_body,
    out_type=jax.ShapeDtypeStruct((HIST,), jnp.float32),
    mesh=plsc.VectorSubcoreMesh(core_axis_name="c", subcore_axis_name="s",
                                num_cores=1),
    scratch_types=[
        pltpu.VMEM((DEG_CPT, DCH), jnp.int32),
        pltpu.VMEM((DCH,), jnp.float32),
        pltpu.VMEM((SLICE,), jnp.float32),
        pltpu.VMEM_SHARED((HIST,), jnp.float32),
        pltpu.SemaphoreType.DMA,
    ],
)


NBUF = 4
NIB = 8


def _edge_body(src_hbm, dst_hbm, y_hbm, out_hbm, sidxb, didxb, rows, agg_sh,
               *sems):
    c = lax.axis_index("c")
    s = lax.axis_index("s")
    wid = c * NS + s
    isems = sems[:NIB]
    gsems = sems[NIB:NIB + NBUF]
    ssems = sems[NIB + NBUF:]

    def zero_rows0(r, _):
        for j in range(D // 16):
            rows[0, r, pl.ds(j * 16, 16)] = jnp.zeros((16,), jnp.float32)
        return 0
    lax.fori_loop(0, CHUNK, zero_rows0, 0)
    for k in range(SLICE // CHUNK):
        pltpu.sync_copy(rows.at[0],
                        agg_sh.at[pl.ds(s * SLICE + k * CHUNK, CHUNK)])
    plsc.subcore_barrier()

    def idx_load(ci, ib):
        pltpu.async_copy(src_hbm.at[wid, ci], sidxb.at[ib], isems[ib])
        pltpu.async_copy(dst_hbm.at[wid, ci], didxb.at[ib], isems[ib])

    def wait_idx(ci, ib):
        pltpu.make_async_copy(src_hbm.at[wid, ci], sidxb.at[ib],
                              isems[ib]).wait()
        pltpu.make_async_copy(dst_hbm.at[wid, ci], didxb.at[ib],
                              isems[ib]).wait()

    def gather(ib, b):
        pltpu.async_copy(y_hbm.at[sidxb.at[ib]], rows.at[b], gsems[b])

    def wait_gather(ib, b):
        pltpu.make_async_copy(y_hbm.at[sidxb.at[ib]], rows.at[b],
                              gsems[b]).wait()

    def scatter(ib, b):
        pltpu.async_copy(rows.at[b], agg_sh.at[didxb.at[ib]], ssems[b],
                         add=True)

    def wait_scatter(ib, b):
        pltpu.make_async_copy(rows.at[b], agg_sh.at[didxb.at[ib]],
                              ssems[b]).wait()

    for k in range(6):
        idx_load(k, k)
    wait_idx(0, 0)
    gather(0, 0)
    wait_idx(1, 1)
    gather(1, 1)

    @pl.loop(0, N_CHUNKS, step=8)
    def octet(g):
        for u in range(8):
            ci = g + u
            b = u % NBUF
            ib = u % NIB
            if u >= 2:
                wait_scatter((u - 2) % NIB, (u - 2) % NBUF)
            else:
                @pl.when(g > 0)
                def _():
                    wait_scatter((u - 2) % NIB, (u - 2) % NBUF)
            if u < 6:
                wait_idx(ci + 2, (u + 2) % NIB)
                gather((u + 2) % NIB, (u + 2) % NBUF)
            else:
                @pl.when(ci + 2 < N_CHUNKS)
                def _():
                    wait_idx(ci + 2, (u + 2) % NIB)
                    gather((u + 2) % NIB, (u + 2) % NBUF)
            if u < 2:
                idx_load(ci + 6, (u + 6) % NIB)
            else:
                @pl.when(ci + 6 < N_CHUNKS)
                def _():
                    idx_load(ci + 6, (u + 6) % NIB)
            wait_gather(ib, b)
            scatter(ib, b)

    wait_scatter((N_CHUNKS - 2) % NIB, (N_CHUNKS - 2) % NBUF)
    wait_scatter((N_CHUNKS - 1) % NIB, (N_CHUNKS - 1) % NBUF)
    plsc.subcore_barrier()
    pltpu.sync_copy(agg_sh.at[pl.ds(s * SLICE, SLICE)],
                    out_hbm.at[c, pl.ds(s * SLICE, SLICE), :])


_edge_kernel = pl.kernel(
    _edge_body,
    out_type=jax.ShapeDtypeStruct((NC, HIST, D), jnp.float32),
    mesh=plsc.VectorSubcoreMesh(core_axis_name="c", subcore_axis_name="s"),
    scratch_types=[
        pltpu.VMEM((NIB, CHUNK), jnp.int32),
        pltpu.VMEM((NIB, CHUNK), jnp.int32),
        pltpu.VMEM((NBUF, CHUNK, D), jnp.float32),
        pltpu.VMEM_SHARED((HIST, D), jnp.float32),
    ] + [pltpu.SemaphoreType.DMA] * (NIB + 2 * NBUF),
)


def _mm_body(x_ref, w_ref, xw_ref):
    xw_ref[...] = jnp.dot(x_ref[...], w_ref[...],
                          preferred_element_type=jnp.float32)


_mm_kernel = pl.pallas_call(
    _mm_body,
    out_shape=jax.ShapeDtypeStruct((N_NODES, D), jnp.float32),
)


def _scale_body(xw_ref, deg_ref, y_ref, dis_ref):
    dis = jax.lax.rsqrt(deg_ref[...] + 1.0)
    dis_ref[...] = dis
    y_ref[...] = xw_ref[...] * dis


_scale_kernel = pl.pallas_call(
    _scale_body,
    out_shape=(
        jax.ShapeDtypeStruct((N_NODES, D), jnp.float32),
        jax.ShapeDtypeStruct((N_NODES, 1), jnp.float32),
    ),
)


def _final_body(p_ref, y_ref, dis_ref, x_ref, b_ref, g_ref, be_ref, o_ref):
    agg = p_ref[0, :N_NODES, :] + p_ref[1, :N_NODES, :] + y_ref[...]
    h = agg * dis_ref[...] + b_ref[...]
    mean = jnp.mean(h, axis=0, keepdims=True)
    cent = h - mean
    var = jnp.mean(cent * cent, axis=0, keepdims=True)
    bn = cent * jax.lax.rsqrt(var + BN_EPS) * g_ref[...] + be_ref[...]
    o_ref[...] = jnp.maximum(bn, 0.0) + x_ref[...]


_final_kernel = pl.pallas_call(
    _final_body,
    out_shape=jax.ShapeDtypeStruct((N_NODES, D), jnp.float32),
)


@jax.jit
def kernel(x, edge_index, W, b, gamma, beta):
    ei = edge_index.astype(jnp.int32)
    pad = E_PAD - ei.shape[1]
    pad_ids = jnp.arange(pad, dtype=jnp.int32)
    src = jnp.concatenate([ei[0], pad_ids % N_NODES])
    dump = DUMP_ROW + pad_ids % (HIST - N_NODES)
    dst = jnp.concatenate([ei[1], dump])

    hist = _deg_kernel(dst.reshape(NS, DEG_CPT, DCH))
    deg_col = hist[:N_NODES].reshape(N_NODES, 1)
    xw = _mm_kernel(x, W)
    y, dis = _scale_kernel(xw, deg_col)
    parts = _edge_kernel(src.reshape(NC * NS, N_CHUNKS, CHUNK),
                         dst.reshape(NC * NS, N_CHUNKS, CHUNK), y)
    return _final_kernel(parts, y, dis, x,
                         b.reshape(1, D), gamma.reshape(1, D),
                         beta.reshape(1, D))


# speedup vs baseline: 40.8077x; 1.0063x over previous
import functools

import jax
import jax.numpy as jnp
from jax import lax
from jax.experimental import pallas as pl
from jax.experimental.pallas import tpu as pltpu
from jax.experimental.pallas import tpu_sc as plsc

N_NODES = 10000
D = 128
BN_EPS = 1e-5

NC = 2
NS = 16
CHUNK = 80
HIST = 10240
SLICE = HIST // NS
DUMP_ROW = N_NODES

E_PER_TILE = 10240
E_PAD = NC * NS * E_PER_TILE
N_CHUNKS = E_PER_TILE // CHUNK

N_EDGES = 320000
E_PAD_TAIL = E_PAD - N_EDGES
REAL_LAST = (N_EDGES - (NC * NS - 1) * E_PER_TILE) // CHUNK

DCH = 128
DEG_CPT = E_PAD // (NS * DCH)


def _zero_vec(ref, n):
    def body(i, _):
        ref[pl.ds(i * 16, 16)] = jnp.zeros((16,), jnp.float32)
        return 0
    lax.fori_loop(0, n // 16, body, 0)


def _zero_rows(ref, rows):
    def body(r, _):
        for j in range(8):
            ref[r, pl.ds(j * 16, 16)] = jnp.zeros((16,), jnp.float32)
        return 0
    lax.fori_loop(0, rows, body, 0)


def _deg_body(dst_hbm, out_hbm, idx_v, ones_v, zero_v, hist_sh, ssem):
    s = lax.axis_index("s")

    def ones_body(i, _):
        ones_v[pl.ds(i * 16, 16)] = jnp.ones((16,), jnp.float32)
        return 0
    lax.fori_loop(0, DCH // 16, ones_body, 0)
    _zero_vec(zero_v, SLICE)
    pltpu.sync_copy(zero_v, hist_sh.at[pl.ds(s * SLICE, SLICE)])
    pltpu.sync_copy(dst_hbm.at[s], idx_v)
    plsc.subcore_barrier()

    def chunk_body(ci, _):
        pltpu.async_copy(ones_v, hist_sh.at[idx_v.at[ci]], ssem, add=True)
        return 0
    lax.fori_loop(0, DEG_CPT, chunk_body, 0)

    def drain_body(ci, _):
        pltpu.make_async_copy(ones_v, hist_sh.at[idx_v.at[ci]], ssem).wait()
        return 0
    lax.fori_loop(0, DEG_CPT, drain_body, 0)

    plsc.subcore_barrier()
    pltpu.sync_copy(hist_sh.at[pl.ds(s * SLICE, SLICE)],
                    out_hbm.at[pl.ds(s * SLICE, SLICE)])


_deg_kernel = pl.kernel(
    _deg_body,
    out_type=jax.ShapeDtypeStruct((HIST,), jnp.float32),
    mesh=plsc.VectorSubcoreMesh(core_axis_name="c", subcore_axis_name="s",
                                num_cores=1),
    scratch_types=[
        pltpu.VMEM((DEG_CPT, DCH), jnp.int32),
        pltpu.VMEM((DCH,), jnp.float32),
        pltpu.VMEM((SLICE,), jnp.float32),
        pltpu.VMEM_SHARED((HIST,), jnp.float32),
        pltpu.SemaphoreType.DMA,
    ],
)


NBUF = 4
NIB = 8


def _edge_body(src_hbm, dst_hbm, y_hbm, out_hbm, sidxb, didxb, rows, agg_sh,
               *sems):
    c = lax.axis_index("c")
    s = lax.axis_index("s")
    wid = c * NS + s
    isems = sems[:NIB]
    gsems = sems[NIB:NIB + NBUF]
    ssems = sems[NIB + NBUF:]

    def zero_rows0(r, _):
        for j in range(D // 16):
            rows[0, r, pl.ds(j * 16, 16)] = jnp.zeros((16,), jnp.float32)
        return 0
    lax.fori_loop(0, CHUNK, zero_rows0, 0)
    for k in range(SLICE // CHUNK):
        pltpu.sync_copy(rows.at[0],
                        agg_sh.at[pl.ds(s * SLICE + k * CHUNK, CHUNK)])
    plsc.subcore_barrier()

    def idx_load(ci, ib):
        pltpu.async_copy(src_hbm.at[wid, ci], sidxb.at[ib], isems[ib])
        pltpu.async_copy(dst_hbm.at[wid, ci], didxb.at[ib], isems[ib])

    def wait_idx(ci, ib):
        pltpu.make_async_copy(src_hbm.at[wid, ci], sidxb.at[ib],
                              isems[ib]).wait()
        pltpu.make_async_copy(dst_hbm.at[wid, ci], didxb.at[ib],
                              isems[ib]).wait()

    def gather(ib, b):
        pltpu.async_copy(y_hbm.at[sidxb.at[ib]], rows.at[b], gsems[b])

    def wait_gather(ib, b):
        pltpu.make_async_copy(y_hbm.at[sidxb.at[ib]], rows.at[b],
                              gsems[b]).wait()

    def scatter(ib, b):
        pltpu.async_copy(rows.at[b], agg_sh.at[didxb.at[ib]], ssems[b],
                         add=True)

    def wait_scatter(ib, b):
        pltpu.make_async_copy(rows.at[b], agg_sh.at[didxb.at[ib]],
                              ssems[b]).wait()

    for k in range(6):
        idx_load(k, k)
    wait_idx(0, 0)
    gather(0, 0)
    wait_idx(1, 1)
    gather(1, 1)

    @pl.loop(0, N_CHUNKS, step=8)
    def octet(g):
        for u in range(8):
            ci = g + u
            b = u % NBUF
            ib = u % NIB
            if u >= 2:
                wait_scatter((u - 2) % NIB, (u - 2) % NBUF)
            else:
                @pl.when(g > 0)
                def _():
                    wait_scatter((u - 2) % NIB, (u - 2) % NBUF)
            if u < 6:
                wait_idx(ci + 2, (u + 2) % NIB)
                gather((u + 2) % NIB, (u + 2) % NBUF)
            else:
                @pl.when(ci + 2 < N_CHUNKS)
                def _():
                    wait_idx(ci + 2, (u + 2) % NIB)
                    gather((u + 2) % NIB, (u + 2) % NBUF)
            if u < 2:
                idx_load(ci + 6, (u + 6) % NIB)
            else:
                @pl.when(ci + 6 < N_CHUNKS)
                def _():
                    idx_load(ci + 6, (u + 6) % NIB)
            wait_gather(ib, b)
            scatter(ib, b)

    wait_scatter((N_CHUNKS - 2) % NIB, (N_CHUNKS - 2) % NBUF)
    wait_scatter((N_CHUNKS - 1) % NIB, (N_CHUNKS - 1) % NBUF)
    plsc.subcore_barrier()
    pltpu.sync_copy(agg_sh.at[pl.ds(s * SLICE, SLICE)],
                    out_hbm.at[c, pl.ds(s * SLICE, SLICE), :])


_edge_kernel = pl.kernel(
    _edge_body,
    out_type=jax.ShapeDtypeStruct((NC, HIST, D), jnp.float32),
    mesh=plsc.VectorSubcoreMesh(core_axis_name="c", subcore_axis_name="s"),
    scratch_types=[
        pltpu.VMEM((NIB, CHUNK), jnp.int32),
        pltpu.VMEM((NIB, CHUNK), jnp.int32),
        pltpu.VMEM((NBUF, CHUNK, D), jnp.float32),
        pltpu.VMEM_SHARED((HIST, D), jnp.float32),
    ] + [pltpu.SemaphoreType.DMA] * (NIB + 2 * NBUF),
)


def _prep_body(x_ref, w_ref, deg_ref, y_ref, dis_ref):
    dis = jax.lax.rsqrt(deg_ref[...] + 1.0)
    dis_ref[...] = dis
    y_ref[...] = jnp.dot(x_ref[...], w_ref[...],
                         preferred_element_type=jnp.float32) * dis


_prep_kernel = pl.pallas_call(
    _prep_body,
    out_shape=(
        jax.ShapeDtypeStruct((N_NODES, D), jnp.float32),
        jax.ShapeDtypeStruct((N_NODES, 1), jnp.float32),
    ),
)


def _final_body(p_ref, y_ref, dis_ref, x_ref, b_ref, g_ref, be_ref, o_ref):
    agg = p_ref[0, :N_NODES, :] + p_ref[1, :N_NODES, :] + y_ref[...]
    h = agg * dis_ref[...] + b_ref[...]
    mean = jnp.mean(h, axis=0, keepdims=True)
    cent = h - mean
    var = jnp.mean(cent * cent, axis=0, keepdims=True)
    bn = cent * jax.lax.rsqrt(var + BN_EPS) * g_ref[...] + be_ref[...]
    o_ref[...] = jnp.maximum(bn, 0.0) + x_ref[...]


_final_kernel = pl.pallas_call(
    _final_body,
    out_shape=jax.ShapeDtypeStruct((N_NODES, D), jnp.float32),
)


@jax.jit
def kernel(x, edge_index, W, b, gamma, beta):
    ei = edge_index.astype(jnp.int32)
    pad = E_PAD - ei.shape[1]
    pad_ids = jnp.arange(pad, dtype=jnp.int32)
    src = jnp.concatenate([ei[0], pad_ids % N_NODES])
    dump = DUMP_ROW + pad_ids % (HIST - N_NODES)
    dst = jnp.concatenate([ei[1], dump])

    hist = _deg_kernel(dst.reshape(NS, DEG_CPT, DCH))
    deg_col = hist[:N_NODES].reshape(N_NODES, 1)
    y, dis = _prep_kernel(x, W, deg_col)
    parts = _edge_kernel(src.reshape(NC * NS, N_CHUNKS, CHUNK),
                         dst.reshape(NC * NS, N_CHUNKS, CHUNK), y)
    return _final_kernel(parts, y, dis, x,
                         b.reshape(1, D), gamma.reshape(1, D),
                         beta.reshape(1, D))
